# baseline (speedup 1.0000x reference)
import functools

import jax
import jax.numpy as jnp
from jax import lax
from jax.experimental import pallas as pl
from jax.experimental.pallas import tpu as pltpu
from jax.experimental.pallas import tpu_sc as plsc

_N = 10000
_E = 320000
_D = 128
_GW = 80
_EB = 2560
_SUBCORES = 16
_CORES = 2
_NP = 10112
_ROWS_PER_SUB = _NP // _SUBCORES

_C = 5
_CW = _E // _C
_GWC = _CW // _GW
_EBC = _CW // _EB

_mesh = plsc.VectorSubcoreMesh(core_axis_name="core", subcore_axis_name="subcore")


_NW = _CORES * _SUBCORES
_WPW = _GWC // _NW
_ROWS_PER_WORKER = _WPW * _GW


def _gather_sc(x, src_idx4, ci):
    @functools.partial(
        pl.kernel,
        mesh=_mesh,
        out_type=jax.ShapeDtypeStruct((_CW, _D), jnp.float32),
        scratch_types=[
            pltpu.VMEM((_WPW, _GW), jnp.int32),
            pltpu.VMEM((_GW, _D), jnp.float32),
            pltpu.VMEM((_GW, _D), jnp.float32),
            pltpu.SemaphoreType.DMA,
            pltpu.SemaphoreType.DMA,
            pltpu.SemaphoreType.DMA,
            pltpu.SemaphoreType.DMA,
        ],
    )
    def k(x_hbm, i_hbm, o_hbm, idx_v, buf0, buf1, gs0, gs1, ws0, ws1):
        cid = lax.axis_index("core")
        sid = lax.axis_index("subcore")
        w = sid * _CORES + cid
        base = w * _ROWS_PER_WORKER
        bufs = (buf0, buf1)
        gsems = (gs0, gs1)
        wsems = (ws0, ws1)

        pltpu.sync_copy(i_hbm.at[ci, w], idx_v)

        gathers = [None] * _WPW
        writes = [None] * _WPW
        gathers[0] = pltpu.async_copy(x_hbm.at[idx_v.at[0]], bufs[0], gsems[0])
        for j in range(_WPW):
            if j + 1 < _WPW:
                if j >= 1:
                    writes[j - 1].wait()
                p = (j + 1) % 2
                gathers[j + 1] = pltpu.async_copy(
                    x_hbm.at[idx_v.at[j + 1]], bufs[p], gsems[p])
            gathers[j].wait()
            writes[j] = pltpu.async_copy(
                bufs[j % 2], o_hbm.at[pl.ds(base + j * _GW, _GW)], wsems[j % 2])
        writes[_WPW - 2].wait()
        writes[_WPW - 1].wait()

    return k(x, src_idx4)


def _scatter_sc(f_c, dst_idx4, init, ci):
    @functools.partial(
        pl.kernel,
        mesh=_mesh,
        out_type=jax.ShapeDtypeStruct((_CORES, _NP, _D), jnp.float32),
        scratch_types=[
            pltpu.VMEM_SHARED((_NP, _D), jnp.float32),
            pltpu.VMEM((_WPW, _GW), jnp.int32),
            pltpu.VMEM((_GW, _D), jnp.float32),
            pltpu.VMEM((_GW, _D), jnp.float32),
            pltpu.SemaphoreType.DMA,
            pltpu.SemaphoreType.DMA,
            pltpu.SemaphoreType.DMA,
            pltpu.SemaphoreType.DMA,
        ],
    )
    def k(f_hbm, i_hbm, p_hbm, o_hbm, acc, idx_v, buf0, buf1, fs0, fs1, ss0, ss1):
        cid = lax.axis_index("core")
        sid = lax.axis_index("subcore")
        w = sid * _CORES + cid
        r0 = sid * _ROWS_PER_SUB
        bufs = (buf0, buf1)
        fsems = (fs0, fs1)
        ssems = (ss0, ss1)

        if ci == 0:
            pltpu.sync_copy(p_hbm.at[pl.ds(r0, _ROWS_PER_SUB)],
                            acc.at[pl.ds(r0, _ROWS_PER_SUB)])
        else:
            pltpu.sync_copy(p_hbm.at[cid, pl.ds(r0, _ROWS_PER_SUB)],
                            acc.at[pl.ds(r0, _ROWS_PER_SUB)])
        pltpu.sync_copy(i_hbm.at[ci, w], idx_v)
        plsc.subcore_barrier()

        base = w * _ROWS_PER_WORKER
        loads = [None] * _WPW
        adds = [None] * _WPW
        loads[0] = pltpu.async_copy(
            f_hbm.at[pl.ds(base, _GW)], bufs[0], fsems[0])
        for j in range(_WPW):
            if j + 1 < _WPW:
                if j >= 1:
                    adds[j - 1].wait()
                p = (j + 1) % 2
                loads[j + 1] = pltpu.async_copy(
                    f_hbm.at[pl.ds(base + (j + 1) * _GW, _GW)], bufs[p], fsems[p])
            loads[j].wait()
            adds[j] = pltpu.async_copy(
                bufs[j % 2], acc.at[idx_v.at[j]], ssems[j % 2], add=True)
        adds[_WPW - 2].wait()
        adds[_WPW - 1].wait()

        plsc.subcore_barrier()
        pltpu.sync_copy(acc.at[pl.ds(r0, _ROWS_PER_SUB)],
                        o_hbm.at[cid, pl.ds(r0, _ROWS_PER_SUB)])

    return k(f_c, dst_idx4, init)


def _edge_tc(src_x_c, edge_attr, bases, w_t, b, ci):
    def body(g_ref, ea_ref, ba_ref, w_ref, b_ref, o_ref):
        z = jnp.dot(g_ref[...] + ea_ref[...], w_ref[...],
                    preferred_element_type=jnp.float32) + b_ref[...]
        gelu = 0.5 * z * (1.0 + lax.erf(z * 0.7071067811865476))
        o_ref[...] = gelu * ba_ref[...]

    return pl.pallas_call(
        body,
        grid=(_EBC,),
        in_specs=[
            pl.BlockSpec((_EB, _D), lambda i: (i, 0)),
            pl.BlockSpec((_EB, _D), lambda i, c=ci: (c * _EBC + i, 0)),
            pl.BlockSpec((_EB, _D), lambda i, c=ci: (c * _EBC + i, 0)),
            pl.BlockSpec((_D, _D), lambda i: (0, 0)),
            pl.BlockSpec((1, _D), lambda i: (0, 0)),
        ],
        out_specs=pl.BlockSpec((_EB, _D), lambda i: (i, 0)),
        out_shape=jax.ShapeDtypeStruct((_CW, _D), jnp.float32),
    )(src_x_c, edge_attr, bases, w_t, b)


def _node_tc(x, partials, w1_t, b1, g1, be1, w2_t, b2, g2, be2):
    def body(x_ref, p_ref, w1_ref, b1_ref, g1_ref, be1_ref,
             w2_ref, b2_ref, g2_ref, be2_ref, o_ref):
        y = x_ref[...] + p_ref[0, :_N, :] + p_ref[1, :_N, :]

        h = jnp.dot(y, w1_ref[...], preferred_element_type=jnp.float32) + b1_ref[...]
        mean = jnp.mean(h, axis=0, keepdims=True)
        var = jnp.mean((h - mean) ** 2, axis=0, keepdims=True)
        h = (h - mean) * lax.rsqrt(var + 1e-5) * g1_ref[...] + be1_ref[...]
        h = jnp.maximum(h, 0.0)

        h = jnp.dot(h, w2_ref[...], preferred_element_type=jnp.float32) + b2_ref[...]
        mean = jnp.mean(h, axis=0, keepdims=True)
        var = jnp.mean((h - mean) ** 2, axis=0, keepdims=True)
        h = (h - mean) * lax.rsqrt(var + 1e-5) * g2_ref[...] + be2_ref[...]
        h = jnp.maximum(h, 0.0)

        o_ref[...] = y + h

    return pl.pallas_call(
        body,
        out_shape=jax.ShapeDtypeStruct((_N, _D), jnp.float32),
    )(x, partials, w1_t, b1, g1, be1, w2_t, b2, g2, be2)


def kernel(x, edge_index, edge_attr, bases, W_pre, b_pre,
           W_f1, b_f1, g1, be1, W_f2, b_f2, g2, be2):
    src_idx4 = edge_index[0].astype(jnp.int32).reshape(_C, _NW, _WPW, _GW)
    dst_idx4 = edge_index[1].astype(jnp.int32).reshape(_C, _NW, _WPW, _GW)

    w_pre_t = W_pre.T
    b_pre_r = b_pre.reshape(1, _D)

    partials = jnp.zeros((_NP, _D), jnp.float32)
    for ci in range(_C):
        g = _gather_sc(x, src_idx4, ci)
        f = _edge_tc(g, edge_attr, bases, w_pre_t, b_pre_r, ci)
        partials = _scatter_sc(f, dst_idx4, partials, ci)
    return _node_tc(x, partials, W_f1.T, b_f1.reshape(1, _D),
                    g1.reshape(1, _D), be1.reshape(1, _D),
                    W_f2.T, b_f2.reshape(1, _D),
                    g2.reshape(1, _D), be2.reshape(1, _D))


# speedup vs baseline: 1.0296x; 1.0296x over previous
import functools

import jax
import jax.numpy as jnp
from jax import lax
from jax.experimental import pallas as pl
from jax.experimental.pallas import tpu as pltpu
from jax.experimental.pallas import tpu_sc as plsc

_N = 10000
_E = 320000
_D = 128
_GW = 80
_EB = 2560
_SUBCORES = 16
_CORES = 2
_NP = 10112
_ROWS_PER_SUB = _NP // _SUBCORES

_C = 5
_CW = _E // _C
_GWC = _CW // _GW
_EBC = _CW // _EB

_mesh = plsc.VectorSubcoreMesh(core_axis_name="core", subcore_axis_name="subcore")


_NW = _CORES * _SUBCORES
_WPW = _GWC // _NW
_ROWS_PER_WORKER = _WPW * _GW


_NBUF = 4


def _gather_sc(x_tab, src_idx4, ci):
    @functools.partial(
        pl.kernel,
        mesh=_mesh,
        out_type=jax.ShapeDtypeStruct((_CW, _D), jnp.float32),
        scratch_types=(
            [pltpu.VMEM((_WPW, _GW), jnp.int32)]
            + [pltpu.VMEM((_GW, _D), jnp.float32)] * _NBUF
            + [pltpu.SemaphoreType.DMA] * (2 * _NBUF)
        ),
    )
    def k(x_hbm, i_hbm, o_hbm, idx_v, *rest):
        bufs = rest[:_NBUF]
        gsems = rest[_NBUF:2 * _NBUF]
        wsems = rest[2 * _NBUF:]
        cid = lax.axis_index("core")
        sid = lax.axis_index("subcore")
        w = sid * _CORES + cid
        base = w * _ROWS_PER_WORKER

        pltpu.sync_copy(i_hbm.at[ci, w], idx_v)

        gathers = [None] * _WPW
        writes = [None] * _WPW

        def start_gather(j):
            gathers[j] = pltpu.async_copy(
                x_hbm.at[idx_v.at[j]], bufs[j % _NBUF], gsems[j % _NBUF])

        for j in range(min(_NBUF - 1, _WPW)):
            start_gather(j)
        for j in range(_WPW):
            k_next = j + _NBUF - 1
            if k_next < _WPW:
                if j >= 1:
                    writes[j - 1].wait()
                start_gather(k_next)
            gathers[j].wait()
            writes[j] = pltpu.async_copy(
                bufs[j % _NBUF], o_hbm.at[pl.ds(base + j * _GW, _GW)],
                wsems[j % _NBUF])
        for j in range(max(0, _WPW - _NBUF), _WPW):
            writes[j].wait()

    return k(x_tab, src_idx4)


def _scatter_sc(fs, dst_idx, init, chunk_ids):
    n = len(fs)

    @functools.partial(
        pl.kernel,
        mesh=_mesh,
        out_type=jax.ShapeDtypeStruct((_CORES, _NP, _D), jnp.float32),
        scratch_types=[pltpu.VMEM_SHARED((_NP, _D), jnp.float32)],
    )
    def k(*refs):
        f_hbms = refs[:n]
        i_hbm, p_hbm, o_hbm, acc = refs[n:]
        cid = lax.axis_index("core")
        sid = lax.axis_index("subcore")
        r0 = sid * _ROWS_PER_SUB
        if init.ndim == 2:
            pltpu.sync_copy(p_hbm.at[pl.ds(r0, _ROWS_PER_SUB)],
                            acc.at[pl.ds(r0, _ROWS_PER_SUB)])
        else:
            pltpu.sync_copy(p_hbm.at[cid, pl.ds(r0, _ROWS_PER_SUB)],
                            acc.at[pl.ds(r0, _ROWS_PER_SUB)])
        plsc.subcore_barrier()

        def body(f_vmem, i_vmem):
            pltpu.sync_copy(f_vmem, acc.at[i_vmem.at[0]], add=True)

        for f_hbm, ci in zip(f_hbms, chunk_ids):
            pltpu.emit_pipeline(
                body,
                grid=(_GWC,),
                in_specs=[pl.BlockSpec((_GW, _D), lambda i: (i, 0)),
                          pl.BlockSpec((1, _GW), lambda i, c=ci: (c * _GWC + i, 0))],
                out_specs=[],
                core_axis_name=("core", "subcore"),
                dimension_semantics=(pltpu.PARALLEL,),
            )(f_hbm, i_hbm)

        plsc.subcore_barrier()
        pltpu.sync_copy(acc.at[pl.ds(r0, _ROWS_PER_SUB)],
                        o_hbm.at[cid, pl.ds(r0, _ROWS_PER_SUB)])

    return k(*fs, dst_idx, init)


def _edge_tc(src_x_c, edge_attr, bases, w_t, b, ci):
    def body(g_ref, ea_ref, ba_ref, w_ref, b_ref, o_ref):
        z = jnp.dot(g_ref[...] + ea_ref[...], w_ref[...],
                    preferred_element_type=jnp.float32) + b_ref[...]
        gelu = 0.5 * z * (1.0 + lax.erf(z * 0.7071067811865476))
        o_ref[...] = gelu * ba_ref[...]

    return pl.pallas_call(
        body,
        grid=(_EBC,),
        in_specs=[
            pl.BlockSpec((_EB, _D), lambda i: (i, 0)),
            pl.BlockSpec((_EB, _D), lambda i, c=ci: (c * _EBC + i, 0)),
            pl.BlockSpec((_EB, _D), lambda i, c=ci: (c * _EBC + i, 0)),
            pl.BlockSpec((_D, _D), lambda i: (0, 0)),
            pl.BlockSpec((1, _D), lambda i: (0, 0)),
        ],
        out_specs=pl.BlockSpec((_EB, _D), lambda i: (i, 0)),
        out_shape=jax.ShapeDtypeStruct((_CW, _D), jnp.float32),
    )(src_x_c, edge_attr, bases, w_t, b)


def _node_tc(x, partials, w1_t, b1, g1, be1, w2_t, b2, g2, be2):
    def body(x_ref, p_ref, w1_ref, b1_ref, g1_ref, be1_ref,
             w2_ref, b2_ref, g2_ref, be2_ref, o_ref):
        y = x_ref[...] + p_ref[0, :_N, :] + p_ref[1, :_N, :]

        h = jnp.dot(y, w1_ref[...], preferred_element_type=jnp.float32) + b1_ref[...]
        mean = jnp.mean(h, axis=0, keepdims=True)
        var = jnp.mean((h - mean) ** 2, axis=0, keepdims=True)
        h = (h - mean) * lax.rsqrt(var + 1e-5) * g1_ref[...] + be1_ref[...]
        h = jnp.maximum(h, 0.0)

        h = jnp.dot(h, w2_ref[...], preferred_element_type=jnp.float32) + b2_ref[...]
        mean = jnp.mean(h, axis=0, keepdims=True)
        var = jnp.mean((h - mean) ** 2, axis=0, keepdims=True)
        h = (h - mean) * lax.rsqrt(var + 1e-5) * g2_ref[...] + be2_ref[...]
        h = jnp.maximum(h, 0.0)

        o_ref[...] = y + h

    return pl.pallas_call(
        body,
        out_shape=jax.ShapeDtypeStruct((_N, _D), jnp.float32),
    )(x, partials, w1_t, b1, g1, be1, w2_t, b2, g2, be2)


def kernel(x, edge_index, edge_attr, bases, W_pre, b_pre,
           W_f1, b_f1, g1, be1, W_f2, b_f2, g2, be2):
    src_idx4 = edge_index[0].astype(jnp.int32).reshape(_C, _NW, _WPW, _GW)
    dst_idx = edge_index[1].astype(jnp.int32).reshape(_E // _GW, _GW)

    w_pre_t = W_pre.T
    b_pre_r = b_pre.reshape(1, _D)

    fs = []
    for ci in range(_C):
        g = _gather_sc(x, src_idx4, ci)
        fs.append(_edge_tc(g, edge_attr, bases, w_pre_t, b_pre_r, ci))

    zeros = jnp.zeros((_NP, _D), jnp.float32)
    partials = _scatter_sc(fs[:4], dst_idx, zeros, range(4))
    partials = _scatter_sc(fs[4:], dst_idx, partials, range(4, _C))
    return _node_tc(x, partials, W_f1.T, b_f1.reshape(1, _D),
                    g1.reshape(1, _D), be1.reshape(1, _D),
                    W_f2.T, b_f2.reshape(1, _D),
                    g2.reshape(1, _D), be2.reshape(1, _D))


# speedup vs baseline: 1.0417x; 1.0118x over previous
import functools

import jax
import jax.numpy as jnp
from jax import lax
from jax.experimental import pallas as pl
from jax.experimental.pallas import tpu as pltpu
from jax.experimental.pallas import tpu_sc as plsc

_N = 10000
_E = 320000
_D = 128
_GW = 80
_EB = 2560
_SUBCORES = 16
_CORES = 2
_NW = _CORES * _SUBCORES
_NP = 10112
_ROWS_PER_SUB = _NP // _SUBCORES

_CUNITS = (16, 23, 30, 40, 16)
_CSTART = tuple(sum(_CUNITS[:i]) for i in range(len(_CUNITS)))
_C = len(_CUNITS)
_SGROUPS = ((0, 1, 2), (3,), (4,))

_NBUF = 4

_mesh = plsc.VectorSubcoreMesh(core_axis_name="core", subcore_axis_name="subcore")


def _gather_sc(x_tab, idx_c, u):
    @functools.partial(
        pl.kernel,
        mesh=_mesh,
        out_type=jax.ShapeDtypeStruct((u * _EB, _D), jnp.float32),
        scratch_types=(
            [pltpu.VMEM((u, _GW), jnp.int32)]
            + [pltpu.VMEM((_GW, _D), jnp.float32)] * _NBUF
            + [pltpu.SemaphoreType.DMA] * (2 * _NBUF)
        ),
    )
    def k(x_hbm, i_hbm, o_hbm, idx_v, *rest):
        bufs = rest[:_NBUF]
        gsems = rest[_NBUF:2 * _NBUF]
        wsems = rest[2 * _NBUF:]
        cid = lax.axis_index("core")
        sid = lax.axis_index("subcore")
        w = sid * _CORES + cid
        base = w * u * _GW

        pltpu.sync_copy(i_hbm.at[w], idx_v)

        gathers = [None] * u
        writes = [None] * u

        def start_gather(j):
            gathers[j] = pltpu.async_copy(
                x_hbm.at[idx_v.at[j]], bufs[j % _NBUF], gsems[j % _NBUF])

        for j in range(min(_NBUF - 1, u)):
            start_gather(j)
        for j in range(u):
            nxt = j + _NBUF - 1
            if nxt < u:
                if j >= 1:
                    writes[j - 1].wait()
                start_gather(nxt)
            gathers[j].wait()
            writes[j] = pltpu.async_copy(
                bufs[j % _NBUF], o_hbm.at[pl.ds(base + j * _GW, _GW)],
                wsems[j % _NBUF])
        for j in range(max(0, u - _NBUF), u):
            writes[j].wait()

    return k(x_tab, idx_c)


def _scatter_sc(fs, dst_idx, init, chunk_ids):
    n = len(fs)

    @functools.partial(
        pl.kernel,
        mesh=_mesh,
        out_type=jax.ShapeDtypeStruct((_CORES, _NP, _D), jnp.float32),
        scratch_types=[pltpu.VMEM_SHARED((_NP, _D), jnp.float32)],
    )
    def k(*refs):
        f_hbms = refs[:n]
        i_hbm, p_hbm, o_hbm, acc = refs[n:]
        cid = lax.axis_index("core")
        sid = lax.axis_index("subcore")
        r0 = sid * _ROWS_PER_SUB
        if init.ndim == 2:
            pltpu.sync_copy(p_hbm.at[pl.ds(r0, _ROWS_PER_SUB)],
                            acc.at[pl.ds(r0, _ROWS_PER_SUB)])
        else:
            pltpu.sync_copy(p_hbm.at[cid, pl.ds(r0, _ROWS_PER_SUB)],
                            acc.at[pl.ds(r0, _ROWS_PER_SUB)])
        plsc.subcore_barrier()

        def body(f_vmem, i_vmem):
            pltpu.sync_copy(f_vmem, acc.at[i_vmem.at[0]], add=True)

        for f_hbm, ci in zip(f_hbms, chunk_ids):
            w0 = _CSTART[ci] * 32
            pltpu.emit_pipeline(
                body,
                grid=(_CUNITS[ci] * 32,),
                in_specs=[pl.BlockSpec((_GW, _D), lambda i: (i, 0)),
                          pl.BlockSpec((1, _GW), lambda i, b=w0: (b + i, 0))],
                out_specs=[],
                core_axis_name=("core", "subcore"),
                dimension_semantics=(pltpu.PARALLEL,),
            )(f_hbm, i_hbm)

        plsc.subcore_barrier()
        pltpu.sync_copy(acc.at[pl.ds(r0, _ROWS_PER_SUB)],
                        o_hbm.at[cid, pl.ds(r0, _ROWS_PER_SUB)])

    return k(*fs, dst_idx, init)


def _edge_tc(src_x_c, edge_attr, bases, w_t, b, ci):
    s, u = _CSTART[ci], _CUNITS[ci]

    def body(g_ref, ea_ref, ba_ref, w_ref, b_ref, o_ref):
        z = jnp.dot(g_ref[...] + ea_ref[...], w_ref[...],
                    preferred_element_type=jnp.float32) + b_ref[...]
        gelu = 0.5 * z * (1.0 + lax.erf(z * 0.7071067811865476))
        o_ref[...] = gelu * ba_ref[...]

    return pl.pallas_call(
        body,
        grid=(u,),
        in_specs=[
            pl.BlockSpec((_EB, _D), lambda i: (i, 0)),
            pl.BlockSpec((_EB, _D), lambda i: (s + i, 0)),
            pl.BlockSpec((_EB, _D), lambda i: (s + i, 0)),
            pl.BlockSpec((_D, _D), lambda i: (0, 0)),
            pl.BlockSpec((1, _D), lambda i: (0, 0)),
        ],
        out_specs=pl.BlockSpec((_EB, _D), lambda i: (i, 0)),
        out_shape=jax.ShapeDtypeStruct((u * _EB, _D), jnp.float32),
    )(src_x_c, edge_attr, bases, w_t, b)


def _node_tc(x, partials, w1_t, b1, g1, be1, w2_t, b2, g2, be2):
    def body(x_ref, p_ref, w1_ref, b1_ref, g1_ref, be1_ref,
             w2_ref, b2_ref, g2_ref, be2_ref, o_ref):
        y = x_ref[...] + p_ref[0, :_N, :] + p_ref[1, :_N, :]

        h = jnp.dot(y, w1_ref[...], preferred_element_type=jnp.float32) + b1_ref[...]
        mean = jnp.mean(h, axis=0, keepdims=True)
        var = jnp.mean((h - mean) ** 2, axis=0, keepdims=True)
        h = (h - mean) * lax.rsqrt(var + 1e-5) * g1_ref[...] + be1_ref[...]
        h = jnp.maximum(h, 0.0)

        h = jnp.dot(h, w2_ref[...], preferred_element_type=jnp.float32) + b2_ref[...]
        mean = jnp.mean(h, axis=0, keepdims=True)
        var = jnp.mean((h - mean) ** 2, axis=0, keepdims=True)
        h = (h - mean) * lax.rsqrt(var + 1e-5) * g2_ref[...] + be2_ref[...]
        h = jnp.maximum(h, 0.0)

        o_ref[...] = y + h

    return pl.pallas_call(
        body,
        out_shape=jax.ShapeDtypeStruct((_N, _D), jnp.float32),
    )(x, partials, w1_t, b1, g1, be1, w2_t, b2, g2, be2)


def kernel(x, edge_index, edge_attr, bases, W_pre, b_pre,
           W_f1, b_f1, g1, be1, W_f2, b_f2, g2, be2):
    src_flat = edge_index[0].astype(jnp.int32)
    dst_idx = edge_index[1].astype(jnp.int32).reshape(_E // _GW, _GW)

    w_pre_t = W_pre.T
    b_pre_r = b_pre.reshape(1, _D)

    fs = []
    for ci in range(_C):
        s, u = _CSTART[ci], _CUNITS[ci]
        idx_c = lax.dynamic_slice(src_flat, (s * _EB,), (u * _EB,)).reshape(
            _NW, u, _GW)
        g = _gather_sc(x, idx_c, u)
        fs.append(_edge_tc(g, edge_attr, bases, w_pre_t, b_pre_r, ci))

    partials = jnp.zeros((_NP, _D), jnp.float32)
    for grp in _SGROUPS:
        partials = _scatter_sc([fs[ci] for ci in grp], dst_idx, partials, grp)

    return _node_tc(x, partials, W_f1.T, b_f1.reshape(1, _D),
                    g1.reshape(1, _D), be1.reshape(1, _D),
                    W_f2.T, b_f2.reshape(1, _D),
                    g2.reshape(1, _D), be2.reshape(1, _D))


# speedup vs baseline: 1.1580x; 1.1116x over previous
import functools

import jax
import jax.numpy as jnp
from jax import lax
from jax.experimental import pallas as pl
from jax.experimental.pallas import tpu as pltpu
from jax.experimental.pallas import tpu_sc as plsc

_N = 10000
_E = 320000
_D = 128
_GW = 80
_EB = 2560
_SUBCORES = 16
_CORES = 2
_NW = _CORES * _SUBCORES
_NP = 10112
_ROWS_PER_SUB = _NP // _SUBCORES

_CUNITS = (16, 23, 30, 40, 16)
_CSTART = tuple(sum(_CUNITS[:i]) for i in range(len(_CUNITS)))
_C = len(_CUNITS)
_SGROUPS = ((0, 1, 2), (3,), (4,))
_GGROUPS = ((0,), (1, 2), (3, 4))

_NBUF = 4

_mesh = plsc.VectorSubcoreMesh(core_axis_name="core", subcore_axis_name="subcore")


def _gather_sc(x_pad, idx_c, u):
    @functools.partial(
        pl.kernel,
        mesh=_mesh,
        out_type=jax.ShapeDtypeStruct((u * _EB, _D), jnp.float32),
        scratch_types=(
            [pltpu.VMEM_SHARED((_NP, _D), jnp.float32),
             pltpu.VMEM((u, _GW), jnp.int32)]
            + [pltpu.VMEM((_GW, _D), jnp.float32)] * _NBUF
            + [pltpu.SemaphoreType.DMA] * (2 * _NBUF)
        ),
    )
    def k(x_hbm, i_hbm, o_hbm, tab, idx_v, *rest):
        bufs = rest[:_NBUF]
        gsems = rest[_NBUF:2 * _NBUF]
        wsems = rest[2 * _NBUF:]
        cid = lax.axis_index("core")
        sid = lax.axis_index("subcore")
        w = sid * _CORES + cid
        base = w * u * _GW
        r0 = sid * _ROWS_PER_SUB

        pltpu.sync_copy(i_hbm.at[w], idx_v)
        pltpu.sync_copy(x_hbm.at[pl.ds(r0, _ROWS_PER_SUB)],
                        tab.at[pl.ds(r0, _ROWS_PER_SUB)])
        plsc.subcore_barrier()

        gathers = [None] * u
        writes = [None] * u

        def start_gather(j):
            gathers[j] = pltpu.async_copy(
                tab.at[idx_v.at[j]], bufs[j % _NBUF], gsems[j % _NBUF])

        for j in range(min(_NBUF - 1, u)):
            start_gather(j)
        for j in range(u):
            nxt = j + _NBUF - 1
            if nxt < u:
                if j >= 1:
                    writes[j - 1].wait()
                start_gather(nxt)
            gathers[j].wait()
            writes[j] = pltpu.async_copy(
                bufs[j % _NBUF], o_hbm.at[pl.ds(base + j * _GW, _GW)],
                wsems[j % _NBUF])
        for j in range(max(0, u - _NBUF), u):
            writes[j].wait()

    return k(x_pad, idx_c)


def _scatter_sc(fs, dst_idx, init, chunk_ids):
    n = len(fs)

    @functools.partial(
        pl.kernel,
        mesh=_mesh,
        out_type=jax.ShapeDtypeStruct((_CORES, _NP, _D), jnp.float32),
        scratch_types=[pltpu.VMEM_SHARED((_NP, _D), jnp.float32)],
    )
    def k(*refs):
        f_hbms = refs[:n]
        i_hbm, p_hbm, o_hbm, acc = refs[n:]
        cid = lax.axis_index("core")
        sid = lax.axis_index("subcore")
        r0 = sid * _ROWS_PER_SUB
        if init.ndim == 2:
            pltpu.sync_copy(p_hbm.at[pl.ds(r0, _ROWS_PER_SUB)],
                            acc.at[pl.ds(r0, _ROWS_PER_SUB)])
        else:
            pltpu.sync_copy(p_hbm.at[cid, pl.ds(r0, _ROWS_PER_SUB)],
                            acc.at[pl.ds(r0, _ROWS_PER_SUB)])
        plsc.subcore_barrier()

        def body(f_vmem, i_vmem):
            pltpu.sync_copy(f_vmem, acc.at[i_vmem.at[0]], add=True)

        for f_hbm, ci in zip(f_hbms, chunk_ids):
            w0 = _CSTART[ci] * 32
            pltpu.emit_pipeline(
                body,
                grid=(_CUNITS[ci] * 32,),
                in_specs=[pl.BlockSpec((_GW, _D), lambda i: (i, 0)),
                          pl.BlockSpec((1, _GW), lambda i, b=w0: (b + i, 0))],
                out_specs=[],
                core_axis_name=("core", "subcore"),
                dimension_semantics=(pltpu.PARALLEL,),
            )(f_hbm, i_hbm)

        plsc.subcore_barrier()
        pltpu.sync_copy(acc.at[pl.ds(r0, _ROWS_PER_SUB)],
                        o_hbm.at[cid, pl.ds(r0, _ROWS_PER_SUB)])

    return k(*fs, dst_idx, init)


def _edge_tc(src_x_span, edge_attr, bases, w_t, b, ci, goff):
    s, u = _CSTART[ci], _CUNITS[ci]

    def body(g_ref, ea_ref, ba_ref, w_ref, b_ref, o_ref):
        z = jnp.dot(g_ref[...] + ea_ref[...], w_ref[...],
                    preferred_element_type=jnp.float32) + b_ref[...]
        gelu = 0.5 * z * (1.0 + lax.erf(z * 0.7071067811865476))
        o_ref[...] = gelu * ba_ref[...]

    return pl.pallas_call(
        body,
        grid=(u,),
        in_specs=[
            pl.BlockSpec((_EB, _D), lambda i: (goff + i, 0)),
            pl.BlockSpec((_EB, _D), lambda i: (s + i, 0)),
            pl.BlockSpec((_EB, _D), lambda i: (s + i, 0)),
            pl.BlockSpec((_D, _D), lambda i: (0, 0)),
            pl.BlockSpec((1, _D), lambda i: (0, 0)),
        ],
        out_specs=pl.BlockSpec((_EB, _D), lambda i: (i, 0)),
        out_shape=jax.ShapeDtypeStruct((u * _EB, _D), jnp.float32),
    )(src_x_span, edge_attr, bases, w_t, b)


def _node_tc(x, partials, w1_t, b1, g1, be1, w2_t, b2, g2, be2):
    def body(x_ref, p_ref, w1_ref, b1_ref, g1_ref, be1_ref,
             w2_ref, b2_ref, g2_ref, be2_ref, o_ref):
        y = x_ref[...] + p_ref[0, :_N, :] + p_ref[1, :_N, :]

        h = jnp.dot(y, w1_ref[...], preferred_element_type=jnp.float32) + b1_ref[...]
        mean = jnp.mean(h, axis=0, keepdims=True)
        var = jnp.mean((h - mean) ** 2, axis=0, keepdims=True)
        h = (h - mean) * lax.rsqrt(var + 1e-5) * g1_ref[...] + be1_ref[...]
        h = jnp.maximum(h, 0.0)

        h = jnp.dot(h, w2_ref[...], preferred_element_type=jnp.float32) + b2_ref[...]
        mean = jnp.mean(h, axis=0, keepdims=True)
        var = jnp.mean((h - mean) ** 2, axis=0, keepdims=True)
        h = (h - mean) * lax.rsqrt(var + 1e-5) * g2_ref[...] + be2_ref[...]
        h = jnp.maximum(h, 0.0)

        o_ref[...] = y + h

    return pl.pallas_call(
        body,
        out_shape=jax.ShapeDtypeStruct((_N, _D), jnp.float32),
    )(x, partials, w1_t, b1, g1, be1, w2_t, b2, g2, be2)


def kernel(x, edge_index, edge_attr, bases, W_pre, b_pre,
           W_f1, b_f1, g1, be1, W_f2, b_f2, g2, be2):
    src_flat = edge_index[0].astype(jnp.int32)
    dst_idx = edge_index[1].astype(jnp.int32).reshape(_E // _GW, _GW)

    w_pre_t = W_pre.T
    b_pre_r = b_pre.reshape(1, _D)

    x_pad = jnp.pad(x, ((0, _NP - _N), (0, 0)))
    fs = [None] * _C
    for grp in _GGROUPS:
        s = _CSTART[grp[0]]
        u = sum(_CUNITS[ci] for ci in grp)
        idx_c = lax.dynamic_slice(src_flat, (s * _EB,), (u * _EB,)).reshape(
            _NW, u, _GW)
        gspan = _gather_sc(x_pad, idx_c, u)
        for ci in grp:
            fs[ci] = _edge_tc(gspan, edge_attr, bases, w_pre_t, b_pre_r,
                              ci, _CSTART[ci] - s)

    partials = jnp.zeros((_NP, _D), jnp.float32)
    for grp in _SGROUPS:
        partials = _scatter_sc([fs[ci] for ci in grp], dst_idx, partials, grp)

    return _node_tc(x, partials, W_f1.T, b_f1.reshape(1, _D),
                    g1.reshape(1, _D), be1.reshape(1, _D),
                    W_f2.T, b_f2.reshape(1, _D),
                    g2.reshape(1, _D), be2.reshape(1, _D))


# speedup vs baseline: 1.1777x; 1.0171x over previous
import functools

import jax
import jax.numpy as jnp
from jax import lax
from jax.experimental import pallas as pl
from jax.experimental.pallas import tpu as pltpu
from jax.experimental.pallas import tpu_sc as plsc

_N = 10000
_E = 320000
_D = 128
_GW = 80
_EB = 2560
_SUBCORES = 16
_CORES = 2
_NW = _CORES * _SUBCORES
_NP = 10112
_ROWS_PER_SUB = _NP // _SUBCORES

_CUNITS = (16, 23, 30, 40, 16)
_CSTART = tuple(sum(_CUNITS[:i]) for i in range(len(_CUNITS)))
_C = len(_CUNITS)
_SGROUPS = ((0, 1), (2, 3), (4,))
_GGROUPS = ((0,), (1, 2), (3, 4))

_NBUF = 4

_mesh = plsc.VectorSubcoreMesh(core_axis_name="core", subcore_axis_name="subcore")


def _gather_sc(x_pad, idx_c, u):
    @functools.partial(
        pl.kernel,
        mesh=_mesh,
        out_type=jax.ShapeDtypeStruct((u * _EB, _D), jnp.float32),
        scratch_types=(
            [pltpu.VMEM_SHARED((_NP, _D), jnp.float32),
             pltpu.VMEM((u, _GW), jnp.int32)]
            + [pltpu.VMEM((_GW, _D), jnp.float32)] * _NBUF
            + [pltpu.SemaphoreType.DMA] * (2 * _NBUF)
        ),
    )
    def k(x_hbm, i_hbm, o_hbm, tab, idx_v, *rest):
        bufs = rest[:_NBUF]
        gsems = rest[_NBUF:2 * _NBUF]
        wsems = rest[2 * _NBUF:]
        cid = lax.axis_index("core")
        sid = lax.axis_index("subcore")
        w = sid * _CORES + cid
        base = w * u * _GW
        r0 = sid * _ROWS_PER_SUB

        pltpu.sync_copy(i_hbm.at[w], idx_v)
        pltpu.sync_copy(x_hbm.at[pl.ds(r0, _ROWS_PER_SUB)],
                        tab.at[pl.ds(r0, _ROWS_PER_SUB)])
        plsc.subcore_barrier()

        gathers = [None] * u
        writes = [None] * u

        def start_gather(j):
            gathers[j] = pltpu.async_copy(
                tab.at[idx_v.at[j]], bufs[j % _NBUF], gsems[j % _NBUF])

        for j in range(min(_NBUF - 1, u)):
            start_gather(j)
        for j in range(u):
            nxt = j + _NBUF - 1
            if nxt < u:
                if j >= 1:
                    writes[j - 1].wait()
                start_gather(nxt)
            gathers[j].wait()
            writes[j] = pltpu.async_copy(
                bufs[j % _NBUF], o_hbm.at[pl.ds(base + j * _GW, _GW)],
                wsems[j % _NBUF])
        for j in range(max(0, u - _NBUF), u):
            writes[j].wait()

    return k(x_pad, idx_c)


def _scatter_sc(fs, dst_idx, init, chunk_ids):
    n = len(fs)

    @functools.partial(
        pl.kernel,
        mesh=_mesh,
        out_type=jax.ShapeDtypeStruct((_CORES, _NP, _D), jnp.float32),
        scratch_types=[pltpu.VMEM_SHARED((_NP, _D), jnp.float32)],
    )
    def k(*refs):
        f_hbms = refs[:n]
        i_hbm, p_hbm, o_hbm, acc = refs[n:]
        cid = lax.axis_index("core")
        sid = lax.axis_index("subcore")
        r0 = sid * _ROWS_PER_SUB
        if init.ndim == 2:
            pltpu.sync_copy(p_hbm.at[pl.ds(r0, _ROWS_PER_SUB)],
                            acc.at[pl.ds(r0, _ROWS_PER_SUB)])
        else:
            pltpu.sync_copy(p_hbm.at[cid, pl.ds(r0, _ROWS_PER_SUB)],
                            acc.at[pl.ds(r0, _ROWS_PER_SUB)])
        plsc.subcore_barrier()

        def body(f_vmem, i_vmem):
            pltpu.sync_copy(f_vmem, acc.at[i_vmem.at[0]], add=True)

        for f_hbm, ci in zip(f_hbms, chunk_ids):
            w0 = _CSTART[ci] * 32
            pltpu.emit_pipeline(
                body,
                grid=(_CUNITS[ci] * 32,),
                in_specs=[pl.BlockSpec((_GW, _D), lambda i: (i, 0)),
                          pl.BlockSpec((1, _GW), lambda i, b=w0: (b + i, 0))],
                out_specs=[],
                core_axis_name=("core", "subcore"),
                dimension_semantics=(pltpu.PARALLEL,),
            )(f_hbm, i_hbm)

        plsc.subcore_barrier()
        pltpu.sync_copy(acc.at[pl.ds(r0, _ROWS_PER_SUB)],
                        o_hbm.at[cid, pl.ds(r0, _ROWS_PER_SUB)])

    return k(*fs, dst_idx, init)


def _edge_tc(src_x_span, edge_attr, bases, w_t, b, ci, goff):
    s, u = _CSTART[ci], _CUNITS[ci]

    def body(g_ref, ea_ref, ba_ref, w_ref, b_ref, o_ref):
        z = jnp.dot(g_ref[...] + ea_ref[...], w_ref[...],
                    preferred_element_type=jnp.float32) + b_ref[...]
        gelu = 0.5 * z * (1.0 + lax.erf(z * 0.7071067811865476))
        o_ref[...] = gelu * ba_ref[...]

    return pl.pallas_call(
        body,
        grid=(u,),
        in_specs=[
            pl.BlockSpec((_EB, _D), lambda i: (goff + i, 0)),
            pl.BlockSpec((_EB, _D), lambda i: (s + i, 0)),
            pl.BlockSpec((_EB, _D), lambda i: (s + i, 0)),
            pl.BlockSpec((_D, _D), lambda i: (0, 0)),
            pl.BlockSpec((1, _D), lambda i: (0, 0)),
        ],
        out_specs=pl.BlockSpec((_EB, _D), lambda i: (i, 0)),
        out_shape=jax.ShapeDtypeStruct((u * _EB, _D), jnp.float32),
    )(src_x_span, edge_attr, bases, w_t, b)


def _node_tc(x, partials, w1_t, b1, g1, be1, w2_t, b2, g2, be2):
    def body(x_ref, p_ref, w1_ref, b1_ref, g1_ref, be1_ref,
             w2_ref, b2_ref, g2_ref, be2_ref, o_ref):
        y = x_ref[...] + p_ref[0, :_N, :] + p_ref[1, :_N, :]

        h = jnp.dot(y, w1_ref[...], preferred_element_type=jnp.float32) + b1_ref[...]
        mean = jnp.mean(h, axis=0, keepdims=True)
        var = jnp.mean((h - mean) ** 2, axis=0, keepdims=True)
        h = (h - mean) * lax.rsqrt(var + 1e-5) * g1_ref[...] + be1_ref[...]
        h = jnp.maximum(h, 0.0)

        h = jnp.dot(h, w2_ref[...], preferred_element_type=jnp.float32) + b2_ref[...]
        mean = jnp.mean(h, axis=0, keepdims=True)
        var = jnp.mean((h - mean) ** 2, axis=0, keepdims=True)
        h = (h - mean) * lax.rsqrt(var + 1e-5) * g2_ref[...] + be2_ref[...]
        h = jnp.maximum(h, 0.0)

        o_ref[...] = y + h

    return pl.pallas_call(
        body,
        out_shape=jax.ShapeDtypeStruct((_N, _D), jnp.float32),
    )(x, partials, w1_t, b1, g1, be1, w2_t, b2, g2, be2)


def kernel(x, edge_index, edge_attr, bases, W_pre, b_pre,
           W_f1, b_f1, g1, be1, W_f2, b_f2, g2, be2):
    src_flat = edge_index[0].astype(jnp.int32)
    dst_idx = edge_index[1].astype(jnp.int32).reshape(_E // _GW, _GW)

    w_pre_t = W_pre.T
    b_pre_r = b_pre.reshape(1, _D)

    x_pad = jnp.pad(x, ((0, _NP - _N), (0, 0)))
    fs = [None] * _C
    for grp in _GGROUPS:
        s = _CSTART[grp[0]]
        u = sum(_CUNITS[ci] for ci in grp)
        idx_c = lax.dynamic_slice(src_flat, (s * _EB,), (u * _EB,)).reshape(
            _NW, u, _GW)
        gspan = _gather_sc(x_pad, idx_c, u)
        for ci in grp:
            fs[ci] = _edge_tc(gspan, edge_attr, bases, w_pre_t, b_pre_r,
                              ci, _CSTART[ci] - s)

    partials = jnp.zeros((_NP, _D), jnp.float32)
    for grp in _SGROUPS:
        partials = _scatter_sc([fs[ci] for ci in grp], dst_idx, partials, grp)

    return _node_tc(x, partials, W_f1.T, b_f1.reshape(1, _D),
                    g1.reshape(1, _D), be1.reshape(1, _D),
                    W_f2.T, b_f2.reshape(1, _D),
                    g2.reshape(1, _D), be2.reshape(1, _D))
